# baseline (speedup 1.0000x reference)
import functools

import jax
import jax.numpy as jnp
from jax import lax
from jax.experimental import pallas as pl
from jax.experimental.pallas import tpu as pltpu
from jax.experimental.pallas import tpu_sc as plsc

N = 100000
L = 200000
K = 4

NC = 2
NS = 16
NW = NC * NS

CA = 6256
CL = 12512
CN = 3136

_mesh = plsc.VectorSubcoreMesh(core_axis_name="c", subcore_axis_name="s")
_params = pltpu.CompilerParams(needs_layout_passes=False)


@functools.partial(
    pl.kernel,
    out_type=jax.ShapeDtypeStruct((N,), jnp.float32),
    mesh=_mesh,
    compiler_params=_params,
    scratch_types=[
        pltpu.VMEM_SHARED((N,), jnp.float32),
        pltpu.VMEM_SHARED((L,), jnp.float32),
        pltpu.VMEM((CL,), jnp.int32),
        pltpu.VMEM((CL,), jnp.int32),
        pltpu.VMEM((CL,), jnp.float32),
        pltpu.VMEM((CL,), jnp.float32),
        pltpu.VMEM((CL,), jnp.float32),
        pltpu.VMEM((CL,), jnp.float32),
        pltpu.VMEM((CN * K,), jnp.int32),
        pltpu.VMEM((CN * K,), jnp.float32),
        pltpu.VMEM((CN,), jnp.float32),
        pltpu.SemaphoreType.DMA,
        pltpu.SemaphoreType.DMA,
        pltpu.SemaphoreType.DMA,
        pltpu.SemaphoreType.DMA,
    ],
)
def _slope_fused(head_hbm, tail_hbm, len_hbm, array_hbm, links_hbm, out_hbm,
                 arr_s, grad_s,
                 head_v, tail_v, len_v, hval_v, tval_v, grad_v,
                 links_v, g_v, out_v,
                 sem0, sem1, sem2, sem3):
    cid = lax.axis_index("c")
    sid = lax.axis_index("s")
    wid = sid * NC + cid

    abase = jnp.minimum(sid * CA, N - CA)
    pltpu.sync_copy(array_hbm.at[pl.ds(abase, CA)], grad_v.at[pl.ds(0, CA)])
    cp_a = pltpu.async_copy(grad_v.at[pl.ds(0, CA)],
                            arr_s.at[pl.ds(abase, CA)], sem3)
    lbase = jnp.minimum(sid * CL, L - CL)
    cp_h = pltpu.async_copy(head_hbm.at[pl.ds(lbase, CL)], head_v, sem0)
    cp_t = pltpu.async_copy(tail_hbm.at[pl.ds(lbase, CL)], tail_v, sem1)
    cp_l = pltpu.async_copy(len_hbm.at[pl.ds(lbase, CL)], len_v, sem2)
    cp_a.wait()
    cp_h.wait()
    cp_t.wait()
    plsc.subcore_barrier()
    g_h = pltpu.async_copy(arr_s.at[head_v], hval_v, sem0)
    g_t = pltpu.async_copy(arr_s.at[tail_v], tval_v, sem1)
    cp_l.wait()
    g_h.wait()
    g_t.wait()

    def body_a(i, carry):
        ds = pl.ds(i * 16, 16)
        grad_v[ds] = (hval_v[ds] - tval_v[ds]) / len_v[ds]
        return carry

    lax.fori_loop(0, CL // 16, body_a, 0)
    pltpu.sync_copy(grad_v, grad_s.at[pl.ds(lbase, CL)])
    plsc.subcore_barrier()

    nbase = jnp.minimum(wid * CN, N - CN)
    pltpu.sync_copy(links_hbm.at[pl.ds(nbase * K, CN * K)], links_v)
    pltpu.async_copy(grad_s.at[links_v], g_v, sem0).wait()

    iota4 = lax.iota(jnp.int32, 16) * K

    def body_b(i, carry):
        idx0 = iota4 + i * (16 * K)
        g0 = plsc.load_gather(g_v, [idx0])
        g1 = plsc.load_gather(g_v, [idx0 + 1])
        g2 = plsc.load_gather(g_v, [idx0 + 2])
        g3 = plsc.load_gather(g_v, [idx0 + 3])
        out_v[pl.ds(i * 16, 16)] = ((g0 + g1) + (g2 + g3)) * 0.25
        return carry

    lax.fori_loop(0, CN // 16, body_b, 0)
    pltpu.sync_copy(out_v, out_hbm.at[pl.ds(nbase, CN)])


def kernel(array, length_of_link, node_at_link_head, node_at_link_tail, links_at_node):
    return _slope_fused(node_at_link_head, node_at_link_tail, length_of_link,
                        array, links_at_node.reshape(-1))


# speedup vs baseline: 2.7848x; 2.7848x over previous
import functools

import jax
import jax.numpy as jnp
from jax import lax
from jax.experimental import pallas as pl
from jax.experimental.pallas import tpu as pltpu
from jax.experimental.pallas import tpu_sc as plsc

N = 100000
L = 200000
K = 4

NC = 2
NS = 16
NW = NC * NS

CA = 6256
CL = 12512
CLH = CL // 2
CN = 3136
CNH = CN // 2

_mesh = plsc.VectorSubcoreMesh(core_axis_name="c", subcore_axis_name="s")
_params = pltpu.CompilerParams(needs_layout_passes=False)


@functools.partial(
    pl.kernel,
    out_type=jax.ShapeDtypeStruct((N,), jnp.float32),
    mesh=_mesh,
    compiler_params=_params,
    scratch_types=[
        pltpu.VMEM_SHARED((N,), jnp.float32),
        pltpu.VMEM_SHARED((L,), jnp.float32),
        pltpu.VMEM((CL,), jnp.int32),
        pltpu.VMEM((CL,), jnp.int32),
        pltpu.VMEM((CL,), jnp.float32),
        pltpu.VMEM((CL,), jnp.float32),
        pltpu.VMEM((CL,), jnp.float32),
        pltpu.VMEM((CL,), jnp.float32),
        [pltpu.VMEM((CN,), jnp.int32) for _ in range(K)],
        [pltpu.VMEM((CN,), jnp.float32) for _ in range(K)],
        pltpu.VMEM((CN,), jnp.float32),
        [pltpu.SemaphoreType.DMA for _ in range(8)],
    ],
)
def _slope_fused(head_hbm, tail_hbm, len_hbm, array_hbm, linksT_hbm, out_hbm,
                 arr_s, grad_s,
                 head_v, tail_v, len_v, hval_v, tval_v, grad_v,
                 links_v, g_v, out_v,
                 sems):
    cid = lax.axis_index("c")
    sid = lax.axis_index("s")
    wid = sid * NC + cid

    abase = jnp.minimum(sid * CA, N - CA)
    pltpu.sync_copy(array_hbm.at[pl.ds(abase, CA)], grad_v.at[pl.ds(0, CA)])
    cp_a = pltpu.async_copy(grad_v.at[pl.ds(0, CA)],
                            arr_s.at[pl.ds(abase, CA)], sems[7])
    lbase = jnp.minimum(sid * CL, L - CL)
    cp_h = pltpu.async_copy(head_hbm.at[pl.ds(lbase, CL)], head_v, sems[0])
    cp_t = pltpu.async_copy(tail_hbm.at[pl.ds(lbase, CL)], tail_v, sems[1])
    cp_l = pltpu.async_copy(len_hbm.at[pl.ds(lbase, CL)], len_v, sems[2])

    nbase = jnp.minimum(wid * CN, N - CN)
    idx_cps = [
        pltpu.async_copy(linksT_hbm.at[pl.ds(j * N + nbase, CN)], links_v[j],
                         sems[3 + j])
        for j in range(K)
    ]

    cp_a.wait()
    cp_h.wait()
    cp_t.wait()
    cp_l.wait()
    plsc.subcore_barrier()

    g_h0 = pltpu.async_copy(arr_s.at[head_v.at[pl.ds(0, CLH)]],
                            hval_v.at[pl.ds(0, CLH)], sems[0])
    g_t0 = pltpu.async_copy(arr_s.at[tail_v.at[pl.ds(0, CLH)]],
                            tval_v.at[pl.ds(0, CLH)], sems[1])
    g_h1 = pltpu.async_copy(arr_s.at[head_v.at[pl.ds(CLH, CLH)]],
                            hval_v.at[pl.ds(CLH, CLH)], sems[2])
    g_t1 = pltpu.async_copy(arr_s.at[tail_v.at[pl.ds(CLH, CLH)]],
                            tval_v.at[pl.ds(CLH, CLH)], sems[7])

    def body_a(i, carry):
        ds = pl.ds(i * 16, 16)
        grad_v[ds] = (hval_v[ds] - tval_v[ds]) / len_v[ds]
        return carry

    g_h0.wait()
    g_t0.wait()
    lax.fori_loop(0, CLH // 16, body_a, 0)
    w0 = pltpu.async_copy(grad_v.at[pl.ds(0, CLH)],
                          grad_s.at[pl.ds(lbase, CLH)], sems[0])
    g_h1.wait()
    g_t1.wait()
    lax.fori_loop(CLH // 16, CL // 16, body_a, 0)
    w0.wait()
    pltpu.sync_copy(grad_v.at[pl.ds(CLH, CLH)],
                    grad_s.at[pl.ds(lbase + CLH, CLH)])
    for cp in idx_cps:
        cp.wait()
    plsc.subcore_barrier()

    gb0 = [
        pltpu.async_copy(grad_s.at[links_v[j].at[pl.ds(0, CNH)]],
                         g_v[j].at[pl.ds(0, CNH)], sems[j])
        for j in range(K)
    ]
    gb1 = [
        pltpu.async_copy(grad_s.at[links_v[j].at[pl.ds(CNH, CNH)]],
                         g_v[j].at[pl.ds(CNH, CNH)], sems[4 + j])
        for j in range(K)
    ]

    def body_b(i, carry):
        ds = pl.ds(i * 16, 16)
        out_v[ds] = ((g_v[0][ds] + g_v[1][ds]) + (g_v[2][ds] + g_v[3][ds])) * 0.25
        return carry

    for cp in gb0:
        cp.wait()
    lax.fori_loop(0, CNH // 16, body_b, 0)
    w1 = pltpu.async_copy(out_v.at[pl.ds(0, CNH)],
                          out_hbm.at[pl.ds(nbase, CNH)], sems[0])
    for cp in gb1:
        cp.wait()
    lax.fori_loop(CNH // 16, CN // 16, body_b, 0)
    w1.wait()
    pltpu.sync_copy(out_v.at[pl.ds(CNH, CNH)],
                    out_hbm.at[pl.ds(nbase + CNH, CNH)])


def kernel(array, length_of_link, node_at_link_head, node_at_link_tail, links_at_node):
    linksT = links_at_node.T.reshape(-1)
    return _slope_fused(node_at_link_head, node_at_link_tail, length_of_link,
                        array, linksT)


# speedup vs baseline: 2.8363x; 1.0185x over previous
import functools

import jax
import jax.numpy as jnp
from jax import lax
from jax.experimental import pallas as pl
from jax.experimental.pallas import tpu as pltpu
from jax.experimental.pallas import tpu_sc as plsc

N = 100000
L = 200000
K = 4

NC = 2
NS = 16
NW = NC * NS

CA = 6256
CL = 12544
CLQ = CL // 4
CN = 3136
CNH = CN // 2

_mesh = plsc.VectorSubcoreMesh(core_axis_name="c", subcore_axis_name="s")
_params = pltpu.CompilerParams(needs_layout_passes=False)


@functools.partial(
    pl.kernel,
    out_type=jax.ShapeDtypeStruct((N,), jnp.float32),
    mesh=_mesh,
    compiler_params=_params,
    scratch_types=[
        pltpu.VMEM_SHARED((N,), jnp.float32),
        pltpu.VMEM_SHARED((L,), jnp.float32),
        pltpu.VMEM((CL,), jnp.int32),
        pltpu.VMEM((CL,), jnp.int32),
        pltpu.VMEM((CL,), jnp.float32),
        pltpu.VMEM((CL,), jnp.float32),
        pltpu.VMEM((CL,), jnp.float32),
        pltpu.VMEM((CL,), jnp.float32),
        [pltpu.VMEM((CN,), jnp.int32) for _ in range(K)],
        [pltpu.VMEM((CN,), jnp.float32) for _ in range(K)],
        pltpu.VMEM((CN,), jnp.float32),
        [pltpu.SemaphoreType.DMA for _ in range(13)],
    ],
)
def _slope_fused(head_hbm, tail_hbm, len_hbm, array_hbm, linksT_hbm, out_hbm,
                 arr_s, grad_s,
                 head_v, tail_v, len_v, hval_v, tval_v, grad_v,
                 links_v, g_v, out_v,
                 sems):
    cid = lax.axis_index("c")
    sid = lax.axis_index("s")
    wid = sid * NC + cid

    lbase = jnp.minimum(sid * CL, L - CL)
    cp_h = pltpu.async_copy(head_hbm.at[pl.ds(lbase, CL)], head_v, sems[8])
    cp_t = pltpu.async_copy(tail_hbm.at[pl.ds(lbase, CL)], tail_v, sems[9])
    cp_l = pltpu.async_copy(len_hbm.at[pl.ds(lbase, CL)], len_v, sems[10])

    nbase = jnp.minimum(wid * CN, N - CN)
    idx_cps = [
        pltpu.async_copy(linksT_hbm.at[pl.ds(j * N + nbase, CN)], links_v[j],
                         sems[4 + j])
        for j in range(K)
    ]

    abase = jnp.minimum(sid * CA, N - CA)
    pltpu.sync_copy(array_hbm.at[pl.ds(abase, CA)], grad_v.at[pl.ds(0, CA)])
    cp_a = pltpu.async_copy(grad_v.at[pl.ds(0, CA)],
                            arr_s.at[pl.ds(abase, CA)], sems[12])

    cp_a.wait()
    cp_h.wait()
    cp_t.wait()
    cp_l.wait()
    plsc.subcore_barrier()

    g_h = [
        pltpu.async_copy(arr_s.at[head_v.at[pl.ds(q * CLQ, CLQ)]],
                         hval_v.at[pl.ds(q * CLQ, CLQ)], sems[q])
        for q in range(4)
    ]
    g_t = [
        pltpu.async_copy(arr_s.at[tail_v.at[pl.ds(q * CLQ, CLQ)]],
                         tval_v.at[pl.ds(q * CLQ, CLQ)], sems[8 + q])
        for q in range(4)
    ]

    def body_a(i, carry):
        ds = pl.ds(i * 16, 16)
        grad_v[ds] = (hval_v[ds] - tval_v[ds]) / len_v[ds]
        return carry

    w_cps = []
    for q in range(4):
        g_h[q].wait()
        g_t[q].wait()
        lax.fori_loop(q * (CLQ // 16), (q + 1) * (CLQ // 16), body_a, 0)
        w_cps.append(pltpu.async_copy(grad_v.at[pl.ds(q * CLQ, CLQ)],
                                      grad_s.at[pl.ds(lbase + q * CLQ, CLQ)],
                                      sems[q]))
    for cp in w_cps:
        cp.wait()
    for cp in idx_cps:
        cp.wait()
    plsc.subcore_barrier()

    gb0 = [
        pltpu.async_copy(grad_s.at[links_v[j].at[pl.ds(0, CNH)]],
                         g_v[j].at[pl.ds(0, CNH)], sems[j])
        for j in range(K)
    ]
    gb1 = [
        pltpu.async_copy(grad_s.at[links_v[j].at[pl.ds(CNH, CNH)]],
                         g_v[j].at[pl.ds(CNH, CNH)], sems[4 + j])
        for j in range(K)
    ]

    def body_b(i, carry):
        ds = pl.ds(i * 16, 16)
        out_v[ds] = ((g_v[0][ds] + g_v[1][ds]) + (g_v[2][ds] + g_v[3][ds])) * 0.25
        return carry

    for cp in gb0:
        cp.wait()
    lax.fori_loop(0, CNH // 16, body_b, 0)
    w1 = pltpu.async_copy(out_v.at[pl.ds(0, CNH)],
                          out_hbm.at[pl.ds(nbase, CNH)], sems[0])
    for cp in gb1:
        cp.wait()
    lax.fori_loop(CNH // 16, CN // 16, body_b, 0)
    w1.wait()
    pltpu.sync_copy(out_v.at[pl.ds(CNH, CNH)],
                    out_hbm.at[pl.ds(nbase + CNH, CNH)])


def kernel(array, length_of_link, node_at_link_head, node_at_link_tail, links_at_node):
    linksT = links_at_node.T.reshape(-1)
    return _slope_fused(node_at_link_head, node_at_link_tail, length_of_link,
                        array, linksT)
